# baseline (speedup 1.0000x reference)
import math
import jax
import jax.numpy as jnp
from jax.experimental import pallas as pl
from jax.experimental.pallas import tpu as pltpu


def _round_up(x, m):
    return (x + m - 1) // m * m


def _hw_kernel(h_ref, w_ref, norm_ref, hw_ref):
    hw = jnp.dot(h_ref[...], w_ref[...], preferred_element_type=jnp.float32)
    hw_ref[...] = (hw * norm_ref[...]).astype(hw_ref.dtype)


def _agg_kernel(a_ref, hw_ref, norm_ref, b_ref, o_ref):
    a = a_ref[...].astype(jnp.bfloat16)
    acc = jnp.dot(a, hw_ref[...], preferred_element_type=jnp.float32)
    o_ref[...] = jnp.maximum(acc * norm_ref[...] + b_ref[...], 0.0).astype(
        o_ref.dtype
    )


def kernel(h, weight, norm, adj, bias, *, tile_hw=512, tile_m=1024):
    N, in_feats = h.shape
    out_feats = weight.shape[1]

    norm = norm.reshape(N, 1).astype(jnp.float32)

    f_pad = _round_up(max(out_feats, 128), 128)
    n_pad = _round_up(N, math.lcm(tile_hw, tile_m))

    w_pad = jnp.zeros((in_feats, f_pad), jnp.float32).at[:, :out_feats].set(weight)
    b_pad = jnp.zeros((1, f_pad), jnp.float32).at[:, :out_feats].set(
        bias.reshape(1, -1)
    )
    if n_pad != N:
        h_p = jnp.zeros((n_pad, in_feats), jnp.float32).at[:N, :].set(h)
        norm_p = jnp.zeros((n_pad, 1), jnp.float32).at[:N, :].set(norm)
        a_p = jnp.zeros((n_pad, n_pad), jnp.float32).at[:N, :N].set(adj)
    else:
        h_p, norm_p, a_p = h.astype(jnp.float32), norm, adj.astype(jnp.float32)

    hw = pl.pallas_call(
        _hw_kernel,
        out_shape=jax.ShapeDtypeStruct((n_pad, f_pad), jnp.bfloat16),
        grid=(n_pad // tile_hw,),
        in_specs=[
            pl.BlockSpec((tile_hw, in_feats), lambda i: (i, 0)),
            pl.BlockSpec((in_feats, f_pad), lambda i: (0, 0)),
            pl.BlockSpec((tile_hw, 1), lambda i: (i, 0)),
        ],
        out_specs=pl.BlockSpec((tile_hw, f_pad), lambda i: (i, 0)),
        compiler_params=pltpu.CompilerParams(dimension_semantics=("parallel",)),
    )(h_p, w_pad, norm_p)

    out_p = pl.pallas_call(
        _agg_kernel,
        out_shape=jax.ShapeDtypeStruct((n_pad, f_pad), jnp.float32),
        grid=(n_pad // tile_m,),
        in_specs=[
            pl.BlockSpec((tile_m, n_pad), lambda i: (i, 0)),
            pl.BlockSpec((n_pad, f_pad), lambda i: (0, 0)),
            pl.BlockSpec((tile_m, 1), lambda i: (i, 0)),
            pl.BlockSpec((1, f_pad), lambda i: (0, 0)),
        ],
        out_specs=pl.BlockSpec((tile_m, f_pad), lambda i: (i, 0)),
        compiler_params=pltpu.CompilerParams(dimension_semantics=("arbitrary",)),
    )(a_p, hw, norm_p, b_pad)

    return out_p[:N, :out_feats]


# speedup vs baseline: 1.1128x; 1.1128x over previous
import functools
import math
import jax
import jax.numpy as jnp
from jax.experimental import pallas as pl
from jax.experimental.pallas import tpu as pltpu


def _round_up(x, m):
    return (x + m - 1) // m * m


def _hw_kernel(h_ref, w_ref, norm_ref, hw_ref):
    hw = jnp.dot(h_ref[...], w_ref[...], preferred_element_type=jnp.float32)
    hw_ref[...] = (hw * norm_ref[...]).astype(hw_ref.dtype)


def _agg_kernel(a0_ref, a1_ref, hw_ref, norm_ref, b_ref, o_ref, *, khalf):
    acc = jnp.dot(
        a0_ref[...].astype(jnp.bfloat16),
        hw_ref[:khalf, :],
        preferred_element_type=jnp.float32,
    )
    acc += jnp.dot(
        a1_ref[...].astype(jnp.bfloat16),
        hw_ref[khalf:, :],
        preferred_element_type=jnp.float32,
    )
    o_ref[...] = jnp.maximum(acc * norm_ref[...] + b_ref[...], 0.0).astype(
        o_ref.dtype
    )


def kernel(h, weight, norm, adj, bias, *, tile_hw=512, tile_m=512):
    N, in_feats = h.shape
    out_feats = weight.shape[1]

    norm = norm.reshape(N, 1).astype(jnp.float32)

    f_pad = _round_up(max(out_feats, 128), 128)
    n_pad = _round_up(N, math.lcm(tile_hw, tile_m))

    w_pad = jnp.zeros((in_feats, f_pad), jnp.float32).at[:, :out_feats].set(weight)
    b_pad = jnp.zeros((1, f_pad), jnp.float32).at[:, :out_feats].set(
        bias.reshape(1, -1)
    )
    if n_pad != N:
        h_p = jnp.zeros((n_pad, in_feats), jnp.float32).at[:N, :].set(h)
        norm_p = jnp.zeros((n_pad, 1), jnp.float32).at[:N, :].set(norm)
        a_p = jnp.zeros((n_pad, n_pad), jnp.float32).at[:N, :N].set(adj)
    else:
        h_p, norm_p, a_p = h.astype(jnp.float32), norm, adj.astype(jnp.float32)

    hw = pl.pallas_call(
        _hw_kernel,
        out_shape=jax.ShapeDtypeStruct((n_pad, f_pad), jnp.bfloat16),
        grid=(n_pad // tile_hw,),
        in_specs=[
            pl.BlockSpec((tile_hw, in_feats), lambda i: (i, 0)),
            pl.BlockSpec((in_feats, f_pad), lambda i: (0, 0)),
            pl.BlockSpec((tile_hw, 1), lambda i: (i, 0)),
        ],
        out_specs=pl.BlockSpec((tile_hw, f_pad), lambda i: (i, 0)),
        compiler_params=pltpu.CompilerParams(dimension_semantics=("parallel",)),
    )(h_p, w_pad, norm_p)

    khalf = n_pad // 2
    out_p = pl.pallas_call(
        functools.partial(_agg_kernel, khalf=khalf),
        out_shape=jax.ShapeDtypeStruct((n_pad, f_pad), jnp.float32),
        grid=(n_pad // tile_m,),
        in_specs=[
            pl.BlockSpec((tile_m, khalf), lambda i: (i, 0)),
            pl.BlockSpec((tile_m, khalf), lambda i: (i, 1)),
            pl.BlockSpec((n_pad, f_pad), lambda i: (0, 0)),
            pl.BlockSpec((tile_m, 1), lambda i: (i, 0)),
            pl.BlockSpec((1, f_pad), lambda i: (0, 0)),
        ],
        out_specs=pl.BlockSpec((tile_m, f_pad), lambda i: (i, 0)),
        compiler_params=pltpu.CompilerParams(dimension_semantics=("parallel",)),
    )(a_p, a_p, hw, norm_p, b_pad)

    return out_p[:N, :out_feats]
